# baseline (speedup 1.0000x reference)
import jax
import jax.numpy as jnp
from jax import lax
from jax.experimental import pallas as pl
from jax.experimental.pallas import tpu as pltpu
from jax.experimental.pallas import tpu_sc as plsc

_EPS = 1e-5
_N1 = 10000
_N2 = 2000
_D = 128
_H = 128

_NC = 2
_NS = 16
_K = 2
_BLK = 1024


def _make_sc_agg(n_pad, rows_total):
    cw = _K * 128
    n_chunks = rows_total // _K // (_NC * _NS)
    assert n_chunks % 2 == 0 and n_chunks >= 2
    stripe = n_pad // _NS
    zb = stripe // 16

    mesh = plsc.VectorSubcoreMesh(core_axis_name="c", subcore_axis_name="s")

    def body(src_hbm, dst_hbm, tab_hbm, agg_out, cnt_out,
             agg_sp, sidx, didx, rows_v, zrow_v, cnt_v, gsem, isem, ssem):
        c = lax.axis_index("c")
        s = lax.axis_index("s")
        w = c * _NS + s

        def fire_idx(ch, slot):
            row = w * n_chunks + ch
            pltpu.async_copy(src_hbm.at[pl.ds(row, 1)], sidx.at[slot],
                             isem)
            pltpu.async_copy(dst_hbm.at[pl.ds(row, 1)], didx.at[slot],
                             isem)

        def wait_idx():
            pltpu.make_async_copy(src_hbm.at[pl.ds(0, 1)], sidx.at[0],
                                  isem).wait()
            pltpu.make_async_copy(dst_hbm.at[pl.ds(0, 1)], didx.at[0],
                                  isem).wait()

        def fire_gather(slot):
            pltpu.async_copy(tab_hbm.at[sidx.at[slot, 0]], rows_v, gsem)

        def wait_gather(slot):
            pltpu.make_async_copy(tab_hbm.at[sidx.at[slot, 0]], rows_v,
                                  gsem).wait()

        fire_idx(0, 0)

        zero16 = jnp.zeros((16,), jnp.float32)
        one16 = jnp.ones((16,), jnp.float32)
        for i in range(16):
            for j in range(_H // 16):
                zrow_v[i, pl.ds(16 * j, 16)] = zero16

        def czero(i, carry):
            cnt_v[pl.ds(i * 16, 16)] = zero16
            return carry

        lax.fori_loop(0, n_pad // 16, czero, 0)

        base = s * stripe
        zcps = [pltpu.async_copy(zrow_v, agg_sp.at[pl.ds(base + i * 16, 16)],
                                 ssem)
                for i in range(zb)]
        for cp in zcps:
            cp.wait()
        plsc.subcore_barrier()

        def chunk(i, carry):
            islot = lax.rem(i, 2)
            wait_idx()

            @pl.when(i + 1 < n_chunks)
            def _pref():
                fire_idx(i + 1, 1 - islot)

            fire_gather(islot)
            wait_gather(islot)
            sc = pltpu.async_copy(rows_v, agg_sp.at[didx.at[islot, 0]],
                                  ssem, add=True)

            for g in range(cw // 16):
                idx16 = didx[islot, 0, pl.ds(g * 16, 16)]
                plsc.addupdate_scatter(cnt_v, [idx16], one16)

            sc.wait()
            return carry

        lax.fori_loop(0, n_chunks, chunk, 0)
        plsc.subcore_barrier()

        pltpu.sync_copy(agg_sp.at[pl.ds(base, stripe)],
                        agg_out.at[c, pl.ds(base, stripe)])
        pltpu.sync_copy(cnt_v, cnt_out.at[c, s])

    return pl.kernel(
        body,
        out_type=[
            jax.ShapeDtypeStruct((_NC, n_pad, _H), jnp.float32),
            jax.ShapeDtypeStruct((_NC, _NS, n_pad), jnp.float32),
        ],
        mesh=mesh,
        scratch_types=[
            pltpu.VMEM_SHARED((n_pad, _H), jnp.float32),
            pltpu.VMEM((2, 1, _K * 128), jnp.int32),
            pltpu.VMEM((2, 1, _K * 128), jnp.int32),
            pltpu.VMEM((_K * 128, _H), jnp.float32),
            pltpu.VMEM((16, _H), jnp.float32),
            pltpu.VMEM((n_pad,), jnp.float32),
            pltpu.SemaphoreType.DMA,
            pltpu.SemaphoreType.DMA,
            pltpu.SemaphoreType.DMA,
        ],
        compiler_params=pltpu.CompilerParams(needs_layout_passes=False,
                                             use_tc_tiling_on_sc=False),
    )


def _make_tc_pre(n_pad, n_valid):
    nb = n_pad // _BLK

    def body(agg_ref, cnt_ref, xd_ref, wl_ref, bl_ref, wr_ref,
             h_ref, sums_ref, sumsq_ref):
        b = pl.program_id(0)
        agg = agg_ref[0] + agg_ref[1]
        cnt = jnp.sum(cnt_ref[...], axis=(0, 1))[:, None]
        inv = 1.0 / jnp.maximum(cnt, 1.0)
        mean = agg * inv
        h = (jnp.dot(mean, wl_ref[...], preferred_element_type=jnp.float32)
             + jnp.dot(xd_ref[...], wr_ref[...],
                       preferred_element_type=jnp.float32)
             + bl_ref[...])
        h = jnp.maximum(h, 0.0)
        rows = lax.broadcasted_iota(jnp.int32, (_BLK, 1), 0) + b * _BLK
        h = jnp.where(rows < n_valid, h, 0.0)
        h_ref[...] = h
        sums_ref[0] = jnp.sum(h, axis=0, keepdims=True)
        sumsq_ref[0] = jnp.sum(h * h, axis=0, keepdims=True)

    return pl.pallas_call(
        body,
        grid=(nb,),
        in_specs=[
            pl.BlockSpec((_NC, _BLK, _H), lambda b: (0, b, 0)),
            pl.BlockSpec((_NC, _NS, _BLK), lambda b: (0, 0, b)),
            pl.BlockSpec((_BLK, _D), lambda b: (b, 0)),
            pl.BlockSpec((_D, _D), lambda b: (0, 0)),
            pl.BlockSpec((1, _D), lambda b: (0, 0)),
            pl.BlockSpec((_D, _D), lambda b: (0, 0)),
        ],
        out_specs=[
            pl.BlockSpec((_BLK, _D), lambda b: (b, 0)),
            pl.BlockSpec((1, 1, _D), lambda b: (b, 0, 0)),
            pl.BlockSpec((1, 1, _D), lambda b: (b, 0, 0)),
        ],
        out_shape=[
            jax.ShapeDtypeStruct((n_pad, _D), jnp.float32),
            jax.ShapeDtypeStruct((nb, 1, _D), jnp.float32),
            jax.ShapeDtypeStruct((nb, 1, _D), jnp.float32),
        ],
    )


def _make_tc_bn(n_pad, n_valid, blk_out):
    nb_stats = n_pad // _BLK
    nb = n_valid // blk_out
    inv_n = 1.0 / float(n_valid)

    def body(h_ref, sums_ref, sumsq_ref, g_ref, bt_ref, out_ref):
        mu = jnp.sum(sums_ref[:, 0, :], axis=0) * inv_n
        ex2 = jnp.sum(sumsq_ref[:, 0, :], axis=0) * inv_n
        var = ex2 - mu * mu
        scale = g_ref[0] * lax.rsqrt(var + _EPS)
        shift = bt_ref[0] - mu * scale
        out_ref[...] = h_ref[...] * scale + shift

    return pl.pallas_call(
        body,
        grid=(nb,),
        in_specs=[
            pl.BlockSpec((blk_out, _D), lambda b: (b, 0)),
            pl.BlockSpec((nb_stats, 1, _D), lambda b: (0, 0, 0)),
            pl.BlockSpec((nb_stats, 1, _D), lambda b: (0, 0, 0)),
            pl.BlockSpec((1, _D), lambda b: (0, 0)),
            pl.BlockSpec((1, _D), lambda b: (0, 0)),
        ],
        out_specs=pl.BlockSpec((blk_out, _D), lambda b: (b, 0)),
        out_shape=jax.ShapeDtypeStruct((n_valid, _D), jnp.float32),
    )


def _pad_edges(edge_index, rows_total, dummy_dst):
    e = edge_index.shape[1]
    pad = rows_total * 128 - e
    src = jnp.concatenate(
        [edge_index[0].astype(jnp.int32), jnp.zeros((pad,), jnp.int32)])
    dst = jnp.concatenate(
        [edge_index[1].astype(jnp.int32),
         jnp.full((pad,), dummy_dst, jnp.int32)])
    return (src.reshape(rows_total // _K, _K * 128),
            dst.reshape(rows_total // _K, _K * 128))


_EROWS0 = 4096
_EROWS1 = 1024
_NPAD0 = 10240
_NPAD1 = 2048

_sc0 = _make_sc_agg(_NPAD0, _EROWS0)
_sc1 = _make_sc_agg(_NPAD1, _EROWS1)
_pre0 = _make_tc_pre(_NPAD0, _N1)
_pre1 = _make_tc_pre(_NPAD1, _N2)
_bn0 = _make_tc_bn(_NPAD0, _N1, 1000)
_bn1 = _make_tc_bn(_NPAD1, _N2, 1000)


def kernel(x, edge_index_l0, edge_index_l1,
           W_l0, b_l0, W_r0, g0, bt0,
           W_l1, b_l1, W_r1, g1, bt1):
    src0, dst0 = _pad_edges(edge_index_l0, _EROWS0, _N1)
    agg0, cnt0 = _sc0(src0, dst0, x)
    hp0, s0, q0 = _pre0(agg0, cnt0, x, W_l0, b_l0.reshape(1, _D), W_r0)
    h0 = _bn0(hp0, s0, q0, g0.reshape(1, _D), bt0.reshape(1, _D))

    src1, dst1 = _pad_edges(edge_index_l1, _EROWS1, _N2)
    agg1, cnt1 = _sc1(src1, dst1, h0)
    hp1, s1, q1 = _pre1(agg1, cnt1, h0, W_l1, b_l1.reshape(1, _D), W_r1)
    return _bn1(hp1, s1, q1, g1.reshape(1, _D), bt1.reshape(1, _D))


# speedup vs baseline: 2.0244x; 2.0244x over previous
import jax
import jax.numpy as jnp
from jax import lax
from jax.experimental import pallas as pl
from jax.experimental.pallas import tpu as pltpu
from jax.experimental.pallas import tpu_sc as plsc

_EPS = 1e-5
_N1 = 10000
_N2 = 2000
_D = 128
_H = 128

_NC = 2
_NS = 16
_K = 2
_BLK = 1024


def _make_sc_agg(n_pad, rows_total):
    cpw = rows_total // (_NC * _NS)
    n_chunks = cpw // _K
    stripe = n_pad // _NS
    zb = stripe // 16

    mesh = plsc.VectorSubcoreMesh(core_axis_name="c", subcore_axis_name="s")

    def body(src_hbm, dst_hbm, table_hbm, agg_out, cnt_out,
             agg_sp, src_v, dst_v, rows_v, zrow_v, cnt_v, sem):
        c = lax.axis_index("c")
        s = lax.axis_index("s")
        w = c * _NS + s

        zero16 = jnp.zeros((16,), jnp.float32)
        one16 = jnp.ones((16,), jnp.float32)
        for i in range(16):
            for j in range(8):
                zrow_v[i, pl.ds(16 * j, 16)] = zero16

        def czero(i, carry):
            cnt_v[pl.ds(i * 16, 16)] = zero16
            return carry

        lax.fori_loop(0, n_pad // 16, czero, 0)

        base = s * stripe

        def zloop(i, carry):
            pltpu.sync_copy(zrow_v, agg_sp.at[pl.ds(base + i * 16, 16)])
            return carry

        lax.fori_loop(0, zb, zloop, 0)
        plsc.subcore_barrier()

        def chunk(i, carry):
            row0 = (w * n_chunks + i) * _K
            pltpu.sync_copy(src_hbm.at[pl.ds(row0, _K)], src_v)
            pltpu.sync_copy(dst_hbm.at[pl.ds(row0, _K)], dst_v)
            cps = [pltpu.async_copy(table_hbm.at[src_v.at[j]], rows_v.at[j],
                                    sem)
                   for j in range(_K)]
            for cp in cps:
                cp.wait()
            for j in range(_K):
                pltpu.sync_copy(rows_v.at[j], agg_sp.at[dst_v.at[j]],
                                add=True)
                for g in range(8):
                    idx16 = dst_v[j, pl.ds(g * 16, 16)]
                    plsc.addupdate_scatter(cnt_v, [idx16], one16)
            return carry

        lax.fori_loop(0, n_chunks, chunk, 0)
        plsc.subcore_barrier()

        pltpu.sync_copy(agg_sp.at[pl.ds(base, stripe)],
                        agg_out.at[c, pl.ds(base, stripe)])
        pltpu.sync_copy(cnt_v, cnt_out.at[c, s])

    return pl.kernel(
        body,
        out_type=[
            jax.ShapeDtypeStruct((_NC, n_pad, _D), jnp.float32),
            jax.ShapeDtypeStruct((_NC, _NS, n_pad), jnp.float32),
        ],
        mesh=mesh,
        scratch_types=[
            pltpu.VMEM_SHARED((n_pad, _D), jnp.float32),
            pltpu.VMEM((_K, 128), jnp.int32),
            pltpu.VMEM((_K, 128), jnp.int32),
            pltpu.VMEM((_K, 128, _D), jnp.float32),
            pltpu.VMEM((16, _D), jnp.float32),
            pltpu.VMEM((n_pad,), jnp.float32),
            pltpu.SemaphoreType.DMA,
        ],
        compiler_params=pltpu.CompilerParams(needs_layout_passes=False),
    )


def _make_tc_pre(n_pad, n_valid):
    nb = n_pad // _BLK

    def body(agg_ref, cnt_ref, xd_ref, wl_ref, bl_ref, wr_ref,
             h_ref, sums_ref, sumsq_ref):
        b = pl.program_id(0)
        agg = agg_ref[0] + agg_ref[1]
        cnt = jnp.sum(cnt_ref[...], axis=(0, 1))[:, None]
        inv = 1.0 / jnp.maximum(cnt, 1.0)
        mean = agg * inv
        h = (jnp.dot(mean, wl_ref[...], preferred_element_type=jnp.float32)
             + jnp.dot(xd_ref[...], wr_ref[...],
                       preferred_element_type=jnp.float32)
             + bl_ref[...])
        h = jnp.maximum(h, 0.0)
        rows = lax.broadcasted_iota(jnp.int32, (_BLK, 1), 0) + b * _BLK
        h = jnp.where(rows < n_valid, h, 0.0)
        h_ref[...] = h
        sums_ref[0] = jnp.sum(h, axis=0, keepdims=True)
        sumsq_ref[0] = jnp.sum(h * h, axis=0, keepdims=True)

    return pl.pallas_call(
        body,
        grid=(nb,),
        in_specs=[
            pl.BlockSpec((_NC, _BLK, _H), lambda b: (0, b, 0)),
            pl.BlockSpec((_NC, _NS, _BLK), lambda b: (0, 0, b)),
            pl.BlockSpec((_BLK, _D), lambda b: (b, 0)),
            pl.BlockSpec((_D, _D), lambda b: (0, 0)),
            pl.BlockSpec((1, _D), lambda b: (0, 0)),
            pl.BlockSpec((_D, _D), lambda b: (0, 0)),
        ],
        out_specs=[
            pl.BlockSpec((_BLK, _D), lambda b: (b, 0)),
            pl.BlockSpec((1, 1, _D), lambda b: (b, 0, 0)),
            pl.BlockSpec((1, 1, _D), lambda b: (b, 0, 0)),
        ],
        out_shape=[
            jax.ShapeDtypeStruct((n_pad, _D), jnp.float32),
            jax.ShapeDtypeStruct((nb, 1, _D), jnp.float32),
            jax.ShapeDtypeStruct((nb, 1, _D), jnp.float32),
        ],
    )


def _make_tc_bn(n_pad, n_valid, blk_out):
    nb_stats = n_pad // _BLK
    nb = n_valid // blk_out
    inv_n = 1.0 / float(n_valid)

    def body(h_ref, sums_ref, sumsq_ref, g_ref, bt_ref, out_ref):
        mu = jnp.sum(sums_ref[:, 0, :], axis=0) * inv_n
        ex2 = jnp.sum(sumsq_ref[:, 0, :], axis=0) * inv_n
        var = ex2 - mu * mu
        scale = g_ref[0] * lax.rsqrt(var + _EPS)
        shift = bt_ref[0] - mu * scale
        out_ref[...] = h_ref[...] * scale + shift

    return pl.pallas_call(
        body,
        grid=(nb,),
        in_specs=[
            pl.BlockSpec((blk_out, _D), lambda b: (b, 0)),
            pl.BlockSpec((nb_stats, 1, _D), lambda b: (0, 0, 0)),
            pl.BlockSpec((nb_stats, 1, _D), lambda b: (0, 0, 0)),
            pl.BlockSpec((1, _D), lambda b: (0, 0)),
            pl.BlockSpec((1, _D), lambda b: (0, 0)),
        ],
        out_specs=pl.BlockSpec((blk_out, _D), lambda b: (b, 0)),
        out_shape=jax.ShapeDtypeStruct((n_valid, _D), jnp.float32),
    )


def _pad_edges(edge_index, rows_total, dummy_dst):
    e = edge_index.shape[1]
    pad = rows_total * 128 - e
    src = jnp.concatenate(
        [edge_index[0].astype(jnp.int32), jnp.zeros((pad,), jnp.int32)])
    dst = jnp.concatenate(
        [edge_index[1].astype(jnp.int32),
         jnp.full((pad,), dummy_dst, jnp.int32)])
    return src.reshape(rows_total, 128), dst.reshape(rows_total, 128)


_EROWS0 = 3968
_EROWS1 = 960
_NPAD0 = 10240
_NPAD1 = 2048

_sc0 = _make_sc_agg(_NPAD0, _EROWS0)
_sc1 = _make_sc_agg(_NPAD1, _EROWS1)
_pre0 = _make_tc_pre(_NPAD0, _N1)
_pre1 = _make_tc_pre(_NPAD1, _N2)
_bn0 = _make_tc_bn(_NPAD0, _N1, 1000)
_bn1 = _make_tc_bn(_NPAD1, _N2, 1000)


def kernel(x, edge_index_l0, edge_index_l1,
           W_l0, b_l0, W_r0, g0, bt0,
           W_l1, b_l1, W_r1, g1, bt1):
    src0, dst0 = _pad_edges(edge_index_l0, _EROWS0, _N1)
    agg0, cnt0 = _sc0(src0, dst0, x)
    hp0, s0, q0 = _pre0(agg0, cnt0, x, W_l0, b_l0.reshape(1, _D), W_r0)
    h0 = _bn0(hp0, s0, q0, g0.reshape(1, _D), bt0.reshape(1, _D))

    src1, dst1 = _pad_edges(edge_index_l1, _EROWS1, _N2)
    agg1, cnt1 = _sc1(src1, dst1, h0)
    hp1, s1, q1 = _pre1(agg1, cnt1, h0, W_l1, b_l1.reshape(1, _D), W_r1)
    return _bn1(hp1, s1, q1, g1.reshape(1, _D), bt1.reshape(1, _D))
